# baseline (speedup 1.0000x reference)
import functools
import math

import jax
import jax.numpy as jnp
from jax import lax
from jax.experimental import pallas as pl
from jax.experimental.pallas import tpu as pltpu
from jax.experimental.pallas import tpu_sc as plsc

VOCAB = 100000
D = 768
B = 4
S = 4096
N = B * S
SCALE = math.sqrt(float(D))

_info = plsc.get_sparse_core_info()
NC = _info.num_cores
NS = _info.num_subcores
NW = NC * NS
S_W = S // NW
R = 32
NCH = S_W // R
LANES = 16
JV = D // LANES


def _sc_embed(x_flat, table, pe_s):
    mesh = plsc.VectorSubcoreMesh(core_axis_name="c", subcore_axis_name="s")

    @functools.partial(
        pl.kernel,
        mesh=mesh,
        out_type=jax.ShapeDtypeStruct((N, D), jnp.float32),
        scratch_types=[
            pltpu.VMEM((B * S_W,), jnp.int32),
            pltpu.VMEM((2, R, D), jnp.float32),
            pltpu.VMEM((R, D), jnp.float32),
            pltpu.SemaphoreType.DMA,
            pltpu.SemaphoreType.DMA,
            pltpu.SemaphoreType.DMA,
        ],
    )
    def k(idx_hbm, table_hbm, pe_hbm, out_hbm,
          idx_v, rows_v, pe_v, g_sem, o0, o1):
        wid = lax.axis_index("s") * NC + lax.axis_index("c")
        sbase = wid * S_W
        o_sem = (o0, o1)

        def drain_out(par):
            pltpu.make_async_copy(
                rows_v.at[par], out_hbm.at[pl.ds(0, R)], o_sem[par]).wait()

        for b in range(B):
            pltpu.sync_copy(
                idx_hbm.at[pl.ds(b * S + sbase, S_W)],
                idx_v.at[pl.ds(b * S_W, S_W)])

        def chunk(sc, _):
            pltpu.sync_copy(pe_hbm.at[pl.ds(sbase + sc * R, R)], pe_v)
            for b in range(B):
                par = b % 2
                if b < 2:
                    pl.when(sc >= 1)(lambda par=par: drain_out(par))
                else:
                    drain_out(par)
                pltpu.async_copy(
                    table_hbm.at[idx_v.at[pl.ds(b * S_W + sc * R, R)]],
                    rows_v.at[par], g_sem).wait()

                @plsc.parallel_loop(0, R, step=1, unroll=2)
                def _(r, par=par):
                    for j in range(JV):
                        sl = pl.ds(j * LANES, LANES)
                        rows_v[par, r, sl] = (
                            rows_v[par, r, sl] * SCALE + pe_v[r, sl])
                pltpu.async_copy(
                    rows_v.at[par],
                    out_hbm.at[pl.ds(b * S + sbase + sc * R, R)], o_sem[par])
            return 0

        lax.fori_loop(0, NCH, chunk, 0)
        drain_out(0)
        drain_out(1)

    return k(x_flat, table, pe_s)


def kernel(x, table, pe):
    out = _sc_embed(x.reshape(N), table, pe[:S])
    return out.reshape(B, S, D)


# speedup vs baseline: 1.0426x; 1.0426x over previous
import functools
import math

import jax
import jax.numpy as jnp
from jax import lax
from jax.experimental import pallas as pl
from jax.experimental.pallas import tpu as pltpu
from jax.experimental.pallas import tpu_sc as plsc

VOCAB = 100000
D = 768
B = 4
S = 4096
N = B * S
SCALE = math.sqrt(float(D))

_info = plsc.get_sparse_core_info()
NC = _info.num_cores
NS = _info.num_subcores
NW = NC * NS
S_W = S // NW
R = 32
NCH = S_W // R
LANES = 16
JV = D // LANES


def _sc_embed(x_flat, table, pe_s):
    mesh = plsc.VectorSubcoreMesh(core_axis_name="c", subcore_axis_name="s")

    @functools.partial(
        pl.kernel,
        mesh=mesh,
        out_type=jax.ShapeDtypeStruct((N, D), jnp.float32),
        scratch_types=[
            pltpu.VMEM((B * S_W,), jnp.int32),
            pltpu.VMEM((2, R, D), jnp.float32),
            pltpu.VMEM((R, D), jnp.float32),
            pltpu.SemaphoreType.DMA,
            pltpu.SemaphoreType.DMA,
            pltpu.SemaphoreType.DMA,
        ],
    )
    def k(idx_hbm, table_hbm, pe_hbm, out_hbm,
          idx_v, rows_v, pe_v, g_sem, o0, o1):
        wid = lax.axis_index("s") * NC + lax.axis_index("c")
        sbase = wid * S_W
        o_sem = (o0, o1)

        def drain_out(par):
            pltpu.make_async_copy(
                rows_v.at[par], out_hbm.at[pl.ds(0, R)], o_sem[par]).wait()

        for b in range(B):
            pltpu.sync_copy(
                idx_hbm.at[pl.ds(b * S + sbase, S_W)],
                idx_v.at[pl.ds(b * S_W, S_W)])

        def chunk(sc, _):
            pltpu.sync_copy(pe_hbm.at[pl.ds(sbase + sc * R, R)], pe_v)
            for b in range(B):
                par = b % 2
                if b < 2:
                    pl.when(sc >= 1)(lambda par=par: drain_out(par))
                else:
                    drain_out(par)
                pltpu.async_copy(
                    table_hbm.at[idx_v.at[pl.ds(b * S_W + sc * R, R)]],
                    rows_v.at[par], g_sem).wait()

                def row(r, _, par=par):
                    for j in range(JV):
                        sl = pl.ds(j * LANES, LANES)
                        rows_v[par, r, sl] = (
                            rows_v[par, r, sl] * SCALE + pe_v[r, sl])
                    return 0

                lax.fori_loop(0, R, row, 0)
                pltpu.async_copy(
                    rows_v.at[par],
                    out_hbm.at[pl.ds(b * S + sbase + sc * R, R)], o_sem[par])
            return 0

        lax.fori_loop(0, NCH, chunk, 0)
        drain_out(0)
        drain_out(1)

    return k(x_flat, table, pe_s)


def kernel(x, table, pe):
    out = _sc_embed(x.reshape(N), table, pe[:S])
    return out.reshape(B, S, D)
